# baseline (speedup 1.0000x reference)
import jax
import numpy as np
import jax.numpy as jnp
from jax import lax
from jax.experimental import pallas as pl
from jax.experimental.pallas import tpu as pltpu
from jax.experimental.pallas import tpu_sc as plsc

T = 4096
NCH = T // 16
NEG = np.float32("-inf")
LN2 = np.float32(0.6931471805599453)
CLAMP = np.float32(-87.0)


def _iota():
    return lax.iota(jnp.int32, 16)


def _bi(x):
    return jnp.full((16,), x, jnp.int32)


def _bf(x):
    return jnp.full((16,), x, jnp.float32)


def _log_1_3(s):
    big = s >= 1.5
    h = jnp.where(big, s * 0.5, s)
    z = (h - 1.0) / (h + 1.0)
    w = z * z
    p = w * np.float32(1.0 / 9.0) + np.float32(1.0 / 7.0)
    p = p * w + np.float32(1.0 / 5.0)
    p = p * w + np.float32(1.0 / 3.0)
    p = p * w + np.float32(1.0)
    r = (z + z) * p
    return jnp.where(big, r + LN2, r)


def _lse3(v1, v2, v3):
    mx = jnp.maximum(v1, jnp.maximum(v2, v3))
    e1 = jnp.exp(jnp.maximum(v1 - mx, CLAMP))
    e2 = jnp.exp(jnp.maximum(v2 - mx, CLAMP))
    e3 = jnp.exp(jnp.maximum(v3 - mx, CLAMP))
    return mx + _log_1_3((e1 + e2) + e3)


def _body(seq_hbm, pieces_hbm, plen_hbm, logp_hbm, out_hbm,
          seqv, codes, tb1, tb2, tb3, pcode, pmask, plenv, lpv, piecv,
          lc1, lc2, lc3, emit, aloc, bloc, nloc, pblk, ash, bsh):
    iota = _iota()
    s_id = lax.axis_index("s")
    c_id = lax.axis_index("c")

    pltpu.sync_copy(seq_hbm, seqv)
    pltpu.sync_copy(pieces_hbm, piecv)
    pltpu.sync_copy(plen_hbm, plenv)
    pltpu.sync_copy(logp_hbm, lpv)

    neg16 = _bf(NEG)
    for k in range(8):
        tb1[pl.ds(k * 16, 16)] = neg16
        tb2[pl.ds(k * 16, 16)] = neg16
    for k in range(32):
        tb3[pl.ds(k * 16, 16)] = neg16
    for h in range(2):
        l = plenv[pl.ds(h * 16, 16)]
        lp = lpv[pl.ds(h * 16, 16)]
        r3 = (_bi(h * 16) + iota) * 3
        p0 = plsc.load_gather(piecv, [r3])
        p1 = plsc.load_gather(piecv, [r3 + 1])
        p2 = plsc.load_gather(piecv, [r3 + 2])
        code = (p0 + jnp.where(l >= 2, p1 * 8, 0)) + jnp.where(l >= 3, p2 * 64, 0)
        mask = (_bi(1) << (l * 3)) - 1
        pcode[pl.ds(h * 16, 16)] = code
        pmask[pl.ds(h * 16, 16)] = mask
        plsc.store_scatter(tb1, [code & 7], lp, mask=l == 1)
        plsc.store_scatter(tb2, [code & 63], lp, mask=l == 2)
        plsc.store_scatter(tb3, [code & 511], lp, mask=l == 3)

    def _codes_body(i, carry):
        j = _bi(i * 16) + iota
        s0 = plsc.load_gather(seqv, [j])
        s1 = plsc.load_gather(seqv, [j + 1])
        s2 = plsc.load_gather(seqv, [j + 2])
        plsc.store_scatter(codes, [j], (s0 + 8 * s1) + 64 * s2)
        return carry

    lax.fori_loop(0, NCH, _codes_body, 0)

    @pl.when(s_id == 0)
    def _():
        def lc_body(i, carry):
            j = _bi(i * 16) + iota
            cj = plsc.load_gather(codes, [j])
            plsc.store_scatter(lc1, [j], plsc.load_gather(tb1, [cj & 7]))
            c2 = plsc.load_gather(codes, [jnp.maximum(j - 1, 0)])
            v2 = plsc.load_gather(tb2, [c2 & 63])
            plsc.store_scatter(lc2, [j], jnp.where(j >= 1, v2, NEG))
            c3 = plsc.load_gather(codes, [jnp.maximum(j - 2, 0)])
            v3 = plsc.load_gather(tb3, [c3])
            plsc.store_scatter(lc3, [j], jnp.where(j >= 2, v3, NEG))
            return carry

        lax.fori_loop(0, NCH, lc_body, 0)

        def scan_body(i, carry):
            a1, a2, a3 = carry
            acc = jnp.zeros((16,), jnp.float32)
            for u in range(16):
                jv = _bi(i * 16 + u)
                v1 = a1 + plsc.load_gather(lc1, [jv])
                v2 = a2 + plsc.load_gather(lc2, [jv])
                v3 = a3 + plsc.load_gather(lc3, [jv])
                an = _lse3(v1, v2, v3)
                acc = jnp.where(iota == u, an, acc)
                a3, a2, a1 = a2, a1, an
            plsc.store_scatter(emit, [_bi(i * 16) + iota], acc)
            return (a1, a2, a3)

        lax.fori_loop(0, NCH, scan_body, (_bf(0.0), neg16, neg16))
        pltpu.sync_copy(emit, ash)

    @pl.when(s_id == 1)
    def _():
        def lc_body(i, carry):
            t = _bi(i * 16) + iota
            ct = plsc.load_gather(codes, [t])
            plsc.store_scatter(lc1, [t], plsc.load_gather(tb1, [ct & 7]))
            v2 = plsc.load_gather(tb2, [ct & 63])
            plsc.store_scatter(lc2, [t], jnp.where(t <= T - 2, v2, NEG))
            v3 = plsc.load_gather(tb3, [ct])
            plsc.store_scatter(lc3, [t], jnp.where(t <= T - 3, v3, NEG))
            return carry

        lax.fori_loop(0, NCH, lc_body, 0)

        def scan_body(i, carry):
            b1, b2, b3 = carry
            c = NCH - 1 - i
            acc = jnp.zeros((16,), jnp.float32)
            for u in range(15, -1, -1):
                tv = _bi(c * 16 + u)
                v1 = plsc.load_gather(lc1, [tv]) + b1
                v2 = plsc.load_gather(lc2, [tv]) + b2
                v3 = plsc.load_gather(lc3, [tv]) + b3
                bn = _lse3(v1, v2, v3)
                acc = jnp.where(iota == u, bn, acc)
                b3, b2, b1 = b2, b1, bn
            plsc.store_scatter(emit, [_bi(c * 16) + iota], acc)
            return (b1, b2, b3)

        lax.fori_loop(0, NCH, scan_body, (_bf(0.0), neg16, neg16))
        emit[pl.ds(T, 16)] = jnp.where(iota == 0, np.float32(0.0), NEG)
        pltpu.sync_copy(emit, bsh)

    plsc.subcore_barrier()

    base = c_id * 2048 + s_id * 128
    s0a = pl.multiple_of(jnp.maximum(base - 8, 0), 8)
    basem = pl.multiple_of(base, 128)
    pltpu.sync_copy(ash.at[pl.ds(s0a, 160)], aloc.at[pl.ds(0, 160)])
    pltpu.sync_copy(bsh.at[pl.ds(basem, 160)], bloc.at[pl.ds(0, 160)])
    pltpu.sync_copy(ash.at[pl.ds(T - 16, 16)], nloc.at[pl.ds(0, 16)])
    normv = plsc.load_gather(nloc, [_bi(15)])
    for ch in range(8):
        tv = _bi(base + ch * 16) + iota
        codev = plsc.load_gather(codes, [tv])
        ja = jnp.maximum(tv - 1 - s0a, 0)
        av = plsc.load_gather(aloc, [ja])
        av = jnp.where(tv == 0, np.float32(0.0), av)

        def pbody(v, carry):
            vv = _bi(v)
            cv = plsc.load_gather(pcode, [vv])
            mk = plsc.load_gather(pmask, [vv])
            ln = plsc.load_gather(plenv, [vv])
            lpb = plsc.load_gather(lpv, [vv])
            match = jnp.logical_and((codev & mk) == cv, tv <= T - ln)
            bv = plsc.load_gather(bloc, [(tv + ln) - base])
            logp_tv = ((av + lpb) + bv) - normv
            pv = jnp.exp(jnp.maximum(logp_tv, CLAMP))
            pv = jnp.where(match, pv, np.float32(0.0))
            plsc.store_scatter(pblk, [(_bi(ch * 16) + iota) * 32 + vv], pv)
            return carry

        lax.fori_loop(0, 32, pbody, 0)
    ob = pl.multiple_of(basem * 32, 4096)
    pltpu.sync_copy(pblk, out_hbm.at[pl.ds(ob, 4096)])


_kernel_call = pl.kernel(
    _body,
    out_type=jax.ShapeDtypeStruct((T * 32,), jnp.float32),
    mesh=plsc.VectorSubcoreMesh(core_axis_name="c", subcore_axis_name="s"),
    compiler_params=pltpu.CompilerParams(needs_layout_passes=False),
    scratch_types=[
        pltpu.VMEM((T + 128,), jnp.int32),
        pltpu.VMEM((T,), jnp.int32),
        pltpu.VMEM((128,), jnp.float32),
        pltpu.VMEM((128,), jnp.float32),
        pltpu.VMEM((512,), jnp.float32),
        pltpu.VMEM((128,), jnp.int32),
        pltpu.VMEM((128,), jnp.int32),
        pltpu.VMEM((128,), jnp.int32),
        pltpu.VMEM((128,), jnp.float32),
        pltpu.VMEM((128,), jnp.int32),
        pltpu.VMEM((T,), jnp.float32),
        pltpu.VMEM((T,), jnp.float32),
        pltpu.VMEM((T,), jnp.float32),
        pltpu.VMEM((T + 128,), jnp.float32),
        pltpu.VMEM((256,), jnp.float32),
        pltpu.VMEM((256,), jnp.float32),
        pltpu.VMEM((128,), jnp.float32),
        pltpu.VMEM((T,), jnp.float32),
        pltpu.VMEM_SHARED((T + 128,), jnp.float32),
        pltpu.VMEM_SHARED((T + 128,), jnp.float32),
    ],
)


def kernel(sequence, pieces, piece_len, log_piece_probs):
    i0 = jnp.int32(0)
    seq_pad = jnp.concatenate([sequence, jnp.full((128,), i0, sequence.dtype)])
    pieces_pad = jnp.concatenate([pieces.reshape(96), jnp.full((32,), i0)])
    plen_pad = jnp.concatenate([piece_len, jnp.full((96,), i0, piece_len.dtype)])
    logp_pad = jnp.concatenate(
        [log_piece_probs, jnp.zeros((96,), log_piece_probs.dtype)])
    out = _kernel_call(seq_pad, pieces_pad, plen_pad, logp_pad)
    return out.reshape(T, 32)


# speedup vs baseline: 1.2668x; 1.2668x over previous
import jax
import numpy as np
import jax.numpy as jnp
from jax import lax
from jax.experimental import pallas as pl
from jax.experimental.pallas import tpu as pltpu
from jax.experimental.pallas import tpu_sc as plsc

T = 4096
NCH = T // 16
NEG = np.float32("-inf")
LN2 = np.float32(0.6931471805599453)
CLAMP = np.float32(-87.0)


def _iota():
    return lax.iota(jnp.int32, 16)


def _bi(x):
    return jnp.full((16,), x, jnp.int32)


def _bf(x):
    return jnp.full((16,), x, jnp.float32)


def _build_log_table():
    idx = np.arange(256)
    lo = (0x3F800000 + (idx << 16)).astype(np.uint32).view(np.float32)
    hi = (0x3F800000 + ((idx + 1) << 16)).astype(np.uint32).view(np.float32)
    m = ((lo.astype(np.float64) + hi.astype(np.float64)) / 2).astype(np.float32)
    md = m.astype(np.float64)
    c0 = np.log(md).astype(np.float32)
    c1 = (1.0 / md).astype(np.float32)
    c2 = (-0.5 / (md * md)).astype(np.float32)
    return np.concatenate([m, c0, c1, c2]).astype(np.float32)


_LOGT = _build_log_table()


def _log_tab(s, lt):
    sb = plsc.bitcast(s, jnp.int32)
    i = lax.shift_right_logical(sb - np.int32(0x3F800000), 16)
    m = plsc.load_gather(lt, [i])
    c0 = plsc.load_gather(lt, [i + 256])
    c1 = plsc.load_gather(lt, [i + 512])
    c2 = plsc.load_gather(lt, [i + 768])
    x = s - m
    return (c2 * x + c1) * x + c0


def _log_1_3(s):
    big = s >= 1.5
    h = jnp.where(big, s * 0.5, s)
    z = (h - 1.0) / (h + 1.0)
    w = z * z
    p = w * np.float32(1.0 / 9.0) + np.float32(1.0 / 7.0)
    p = p * w + np.float32(1.0 / 5.0)
    p = p * w + np.float32(1.0 / 3.0)
    p = p * w + np.float32(1.0)
    r = (z + z) * p
    return jnp.where(big, r + LN2, r)


def _lse3(v1, v2, v3, lt):
    mx = jnp.maximum(v1, jnp.maximum(v2, v3))
    e1 = jnp.exp(v1 - mx)
    e2 = jnp.exp(jnp.maximum(v2 - mx, CLAMP))
    e3 = jnp.exp(jnp.maximum(v3 - mx, CLAMP))
    return mx + _log_tab((e1 + e2) + e3, lt)


def _body(seq_hbm, pieces_hbm, plen_hbm, logp_hbm, logt_hbm, out_hbm,
          seqv, codes, tb1, tb2, tb3, pcode, pmask, plenv, lpv, piecv,
          lc1, lc2, lc3, emit, aloc, bloc, nloc, pblk, ltv, ash, bsh):
    iota = _iota()
    s_id = lax.axis_index("s")
    c_id = lax.axis_index("c")

    pltpu.sync_copy(seq_hbm, seqv)
    pltpu.sync_copy(pieces_hbm, piecv)
    pltpu.sync_copy(plen_hbm, plenv)
    pltpu.sync_copy(logp_hbm, lpv)
    pltpu.sync_copy(logt_hbm, ltv)

    neg16 = _bf(NEG)
    for k in range(8):
        tb1[pl.ds(k * 16, 16)] = neg16
        tb2[pl.ds(k * 16, 16)] = neg16
    for k in range(32):
        tb3[pl.ds(k * 16, 16)] = neg16
    for h in range(2):
        l = plenv[pl.ds(h * 16, 16)]
        lp = lpv[pl.ds(h * 16, 16)]
        r3 = (_bi(h * 16) + iota) * 3
        p0 = plsc.load_gather(piecv, [r3])
        p1 = plsc.load_gather(piecv, [r3 + 1])
        p2 = plsc.load_gather(piecv, [r3 + 2])
        code = (p0 + jnp.where(l >= 2, p1 * 8, 0)) + jnp.where(l >= 3, p2 * 64, 0)
        mask = (_bi(1) << (l * 3)) - 1
        pcode[pl.ds(h * 16, 16)] = code
        pmask[pl.ds(h * 16, 16)] = mask
        plsc.store_scatter(tb1, [code & 7], lp, mask=l == 1)
        plsc.store_scatter(tb2, [code & 63], lp, mask=l == 2)
        plsc.store_scatter(tb3, [code & 511], lp, mask=l == 3)

    def _codes_body(i, carry):
        j = _bi(i * 16) + iota
        s0 = plsc.load_gather(seqv, [j])
        s1 = plsc.load_gather(seqv, [j + 1])
        s2 = plsc.load_gather(seqv, [j + 2])
        plsc.store_scatter(codes, [j], (s0 + 8 * s1) + 64 * s2)
        return carry

    lax.fori_loop(0, NCH, _codes_body, 0)

    @pl.when(s_id == 0)
    def _():
        def lc_body(i, carry):
            j = _bi(i * 16) + iota
            cj = plsc.load_gather(codes, [j])
            plsc.store_scatter(lc1, [j], plsc.load_gather(tb1, [cj & 7]))
            c2 = plsc.load_gather(codes, [jnp.maximum(j - 1, 0)])
            v2 = plsc.load_gather(tb2, [c2 & 63])
            plsc.store_scatter(lc2, [j], jnp.where(j >= 1, v2, NEG))
            c3 = plsc.load_gather(codes, [jnp.maximum(j - 2, 0)])
            v3 = plsc.load_gather(tb3, [c3])
            plsc.store_scatter(lc3, [j], jnp.where(j >= 2, v3, NEG))
            return carry

        lax.fori_loop(0, NCH, lc_body, 0)

        def scan_body(i, carry):
            a1, a2, a3 = carry
            acc = jnp.zeros((16,), jnp.float32)
            for u in range(16):
                jv = _bi(i * 16 + u)
                v1 = a1 + plsc.load_gather(lc1, [jv])
                v2 = a2 + plsc.load_gather(lc2, [jv])
                v3 = a3 + plsc.load_gather(lc3, [jv])
                an = _lse3(v1, v2, v3, ltv)
                acc = jnp.where(iota == u, an, acc)
                a3, a2, a1 = a2, a1, an
            plsc.store_scatter(emit, [_bi(i * 16) + iota], acc)
            return (a1, a2, a3)

        lax.fori_loop(0, NCH, scan_body, (_bf(0.0), neg16, neg16))
        pltpu.sync_copy(emit, ash)

    @pl.when(s_id == 1)
    def _():
        def lc_body(i, carry):
            t = _bi(i * 16) + iota
            ct = plsc.load_gather(codes, [t])
            plsc.store_scatter(lc1, [t], plsc.load_gather(tb1, [ct & 7]))
            v2 = plsc.load_gather(tb2, [ct & 63])
            plsc.store_scatter(lc2, [t], jnp.where(t <= T - 2, v2, NEG))
            v3 = plsc.load_gather(tb3, [ct])
            plsc.store_scatter(lc3, [t], jnp.where(t <= T - 3, v3, NEG))
            return carry

        lax.fori_loop(0, NCH, lc_body, 0)

        def scan_body(i, carry):
            b1, b2, b3 = carry
            c = NCH - 1 - i
            acc = jnp.zeros((16,), jnp.float32)
            for u in range(15, -1, -1):
                tv = _bi(c * 16 + u)
                v1 = plsc.load_gather(lc1, [tv]) + b1
                v2 = plsc.load_gather(lc2, [tv]) + b2
                v3 = plsc.load_gather(lc3, [tv]) + b3
                bn = _lse3(v1, v2, v3, ltv)
                acc = jnp.where(iota == u, bn, acc)
                b3, b2, b1 = b2, b1, bn
            plsc.store_scatter(emit, [_bi(c * 16) + iota], acc)
            return (b1, b2, b3)

        lax.fori_loop(0, NCH, scan_body, (_bf(0.0), neg16, neg16))
        emit[pl.ds(T, 16)] = jnp.where(iota == 0, np.float32(0.0), NEG)
        pltpu.sync_copy(emit, bsh)

    plsc.subcore_barrier()

    base = c_id * 2048 + s_id * 128
    s0a = pl.multiple_of(jnp.maximum(base - 8, 0), 8)
    basem = pl.multiple_of(base, 128)
    pltpu.sync_copy(ash.at[pl.ds(s0a, 160)], aloc.at[pl.ds(0, 160)])
    pltpu.sync_copy(bsh.at[pl.ds(basem, 160)], bloc.at[pl.ds(0, 160)])
    pltpu.sync_copy(ash.at[pl.ds(T - 16, 16)], nloc.at[pl.ds(0, 16)])
    normv = plsc.load_gather(nloc, [_bi(15)])
    for ch in range(8):
        tv = _bi(base + ch * 16) + iota
        codev = plsc.load_gather(codes, [tv])
        ja = jnp.maximum(tv - 1 - s0a, 0)
        av = plsc.load_gather(aloc, [ja])
        av = jnp.where(tv == 0, np.float32(0.0), av)

        def pbody(v, carry):
            vv = _bi(v)
            cv = plsc.load_gather(pcode, [vv])
            mk = plsc.load_gather(pmask, [vv])
            ln = plsc.load_gather(plenv, [vv])
            lpb = plsc.load_gather(lpv, [vv])
            match = jnp.logical_and((codev & mk) == cv, tv <= T - ln)
            bv = plsc.load_gather(bloc, [(tv + ln) - base])
            logp_tv = ((av + lpb) + bv) - normv
            pv = jnp.exp(jnp.maximum(logp_tv, CLAMP))
            pv = jnp.where(match, pv, np.float32(0.0))
            plsc.store_scatter(pblk, [(_bi(ch * 16) + iota) * 32 + vv], pv)
            return carry

        lax.fori_loop(0, 32, pbody, 0)
    ob = pl.multiple_of(basem * 32, 4096)
    pltpu.sync_copy(pblk, out_hbm.at[pl.ds(ob, 4096)])


_kernel_call = pl.kernel(
    _body,
    out_type=jax.ShapeDtypeStruct((T * 32,), jnp.float32),
    mesh=plsc.VectorSubcoreMesh(core_axis_name="c", subcore_axis_name="s"),
    compiler_params=pltpu.CompilerParams(needs_layout_passes=False),
    scratch_types=[
        pltpu.VMEM((T + 128,), jnp.int32),
        pltpu.VMEM((T,), jnp.int32),
        pltpu.VMEM((128,), jnp.float32),
        pltpu.VMEM((128,), jnp.float32),
        pltpu.VMEM((512,), jnp.float32),
        pltpu.VMEM((128,), jnp.int32),
        pltpu.VMEM((128,), jnp.int32),
        pltpu.VMEM((128,), jnp.int32),
        pltpu.VMEM((128,), jnp.float32),
        pltpu.VMEM((128,), jnp.int32),
        pltpu.VMEM((T,), jnp.float32),
        pltpu.VMEM((T,), jnp.float32),
        pltpu.VMEM((T,), jnp.float32),
        pltpu.VMEM((T + 128,), jnp.float32),
        pltpu.VMEM((256,), jnp.float32),
        pltpu.VMEM((256,), jnp.float32),
        pltpu.VMEM((128,), jnp.float32),
        pltpu.VMEM((T,), jnp.float32),
        pltpu.VMEM((1024,), jnp.float32),
        pltpu.VMEM_SHARED((T + 128,), jnp.float32),
        pltpu.VMEM_SHARED((T + 128,), jnp.float32),
    ],
)


def kernel(sequence, pieces, piece_len, log_piece_probs):
    i0 = jnp.int32(0)
    seq_pad = jnp.concatenate([sequence, jnp.full((128,), i0, sequence.dtype)])
    pieces_pad = jnp.concatenate([pieces.reshape(96), jnp.full((32,), i0)])
    plen_pad = jnp.concatenate([piece_len, jnp.full((96,), i0, piece_len.dtype)])
    logp_pad = jnp.concatenate(
        [log_piece_probs, jnp.zeros((96,), log_piece_probs.dtype)])
    out = _kernel_call(seq_pad, pieces_pad, plen_pad, logp_pad,
                       jnp.asarray(_LOGT))
    return out.reshape(T, 32)


# speedup vs baseline: 1.2860x; 1.0152x over previous
import jax
import numpy as np
import jax.numpy as jnp
from jax import lax
from jax.experimental import pallas as pl
from jax.experimental.pallas import tpu as pltpu
from jax.experimental.pallas import tpu_sc as plsc

T = 4096
NCH = T // 16
NEG = np.float32("-inf")
LN2 = np.float32(0.6931471805599453)
CLAMP = np.float32(-87.0)


def _iota():
    return lax.iota(jnp.int32, 16)


def _bi(x):
    return jnp.full((16,), x, jnp.int32)


def _bf(x):
    return jnp.full((16,), x, jnp.float32)


def _build_log_table():
    idx = np.arange(256)
    lo = (0x3F800000 + (idx << 16)).astype(np.uint32).view(np.float32)
    hi = (0x3F800000 + ((idx + 1) << 16)).astype(np.uint32).view(np.float32)
    m = ((lo.astype(np.float64) + hi.astype(np.float64)) / 2).astype(np.float32)
    md = m.astype(np.float64)
    c0 = np.log(md).astype(np.float32)
    c1 = (1.0 / md).astype(np.float32)
    c2 = (-0.5 / (md * md)).astype(np.float32)
    return np.concatenate([m, c0, c1, c2]).astype(np.float32)


_LOGT = _build_log_table()


def _log_tab(s, lt):
    sb = plsc.bitcast(s, jnp.int32)
    i = lax.shift_right_logical(sb - np.int32(0x3F800000), 16)
    m = plsc.bitcast((sb & np.int32(~0xFFFF)) | np.int32(0x8000), jnp.float32)
    c0 = plsc.load_gather(lt, [i + 256])
    c1 = plsc.load_gather(lt, [i + 512])
    c2 = plsc.load_gather(lt, [i + 768])
    x = s - m
    return ((c2 * x) * x + (c1 * x + c0))


def _log_1_3(s):
    big = s >= 1.5
    h = jnp.where(big, s * 0.5, s)
    z = (h - 1.0) / (h + 1.0)
    w = z * z
    p = w * np.float32(1.0 / 9.0) + np.float32(1.0 / 7.0)
    p = p * w + np.float32(1.0 / 5.0)
    p = p * w + np.float32(1.0 / 3.0)
    p = p * w + np.float32(1.0)
    r = (z + z) * p
    return jnp.where(big, r + LN2, r)


def _lse3(v1, v2, v3, lt):
    mx = jnp.maximum(v1, jnp.maximum(v2, v3))
    e1 = jnp.exp(v1 - mx)
    e2 = jnp.exp(jnp.maximum(v2 - mx, CLAMP))
    e3 = jnp.exp(jnp.maximum(v3 - mx, CLAMP))
    return mx + _log_tab((e1 + e2) + e3, lt)


def _body(seq_hbm, pieces_hbm, plen_hbm, logp_hbm, logt_hbm, out_hbm,
          seqv, codes, tb1, tb2, tb3, pcode, pmask, plenv, lpv, piecv,
          lc1, lc2, lc3, emit, aloc, bloc, nloc, pblk, ltv, ash, bsh):
    iota = _iota()
    s_id = lax.axis_index("s")
    c_id = lax.axis_index("c")

    pltpu.sync_copy(seq_hbm, seqv)
    pltpu.sync_copy(pieces_hbm, piecv)
    pltpu.sync_copy(plen_hbm, plenv)
    pltpu.sync_copy(logp_hbm, lpv)
    pltpu.sync_copy(logt_hbm, ltv)

    neg16 = _bf(NEG)
    for k in range(8):
        tb1[pl.ds(k * 16, 16)] = neg16
        tb2[pl.ds(k * 16, 16)] = neg16
    for k in range(32):
        tb3[pl.ds(k * 16, 16)] = neg16
    for h in range(2):
        l = plenv[pl.ds(h * 16, 16)]
        lp = lpv[pl.ds(h * 16, 16)]
        r3 = (_bi(h * 16) + iota) * 3
        p0 = plsc.load_gather(piecv, [r3])
        p1 = plsc.load_gather(piecv, [r3 + 1])
        p2 = plsc.load_gather(piecv, [r3 + 2])
        code = (p0 + jnp.where(l >= 2, p1 * 8, 0)) + jnp.where(l >= 3, p2 * 64, 0)
        mask = (_bi(1) << (l * 3)) - 1
        pcode[pl.ds(h * 16, 16)] = code
        pmask[pl.ds(h * 16, 16)] = mask
        plsc.store_scatter(tb1, [code & 7], lp, mask=l == 1)
        plsc.store_scatter(tb2, [code & 63], lp, mask=l == 2)
        plsc.store_scatter(tb3, [code & 511], lp, mask=l == 3)

    def _codes_body(i, carry):
        j = _bi(i * 16) + iota
        s0 = plsc.load_gather(seqv, [j])
        s1 = plsc.load_gather(seqv, [j + 1])
        s2 = plsc.load_gather(seqv, [j + 2])
        plsc.store_scatter(codes, [j], (s0 + 8 * s1) + 64 * s2)
        return carry

    lax.fori_loop(0, NCH, _codes_body, 0)

    @pl.when(s_id == 0)
    def _():
        def lc_body(i, carry):
            j = _bi(i * 16) + iota
            cj = plsc.load_gather(codes, [j])
            plsc.store_scatter(lc1, [j], plsc.load_gather(tb1, [cj & 7]))
            c2 = plsc.load_gather(codes, [jnp.maximum(j - 1, 0)])
            v2 = plsc.load_gather(tb2, [c2 & 63])
            plsc.store_scatter(lc2, [j], jnp.where(j >= 1, v2, NEG))
            c3 = plsc.load_gather(codes, [jnp.maximum(j - 2, 0)])
            v3 = plsc.load_gather(tb3, [c3])
            plsc.store_scatter(lc3, [j], jnp.where(j >= 2, v3, NEG))
            return carry

        lax.fori_loop(0, NCH, lc_body, 0)

        def scan_body(i, carry):
            a1, a2, a3 = carry
            acc = jnp.zeros((16,), jnp.float32)
            for u in range(16):
                jv = _bi(i * 16 + u)
                v1 = a1 + plsc.load_gather(lc1, [jv])
                v2 = a2 + plsc.load_gather(lc2, [jv])
                v3 = a3 + plsc.load_gather(lc3, [jv])
                an = _lse3(v1, v2, v3, ltv)
                acc = jnp.where(iota == u, an, acc)
                a3, a2, a1 = a2, a1, an
            plsc.store_scatter(emit, [_bi(i * 16) + iota], acc)
            return (a1, a2, a3)

        lax.fori_loop(0, NCH, scan_body, (_bf(0.0), neg16, neg16))
        pltpu.sync_copy(emit, ash)

    @pl.when(s_id == 1)
    def _():
        def lc_body(i, carry):
            t = _bi(i * 16) + iota
            ct = plsc.load_gather(codes, [t])
            plsc.store_scatter(lc1, [t], plsc.load_gather(tb1, [ct & 7]))
            v2 = plsc.load_gather(tb2, [ct & 63])
            plsc.store_scatter(lc2, [t], jnp.where(t <= T - 2, v2, NEG))
            v3 = plsc.load_gather(tb3, [ct])
            plsc.store_scatter(lc3, [t], jnp.where(t <= T - 3, v3, NEG))
            return carry

        lax.fori_loop(0, NCH, lc_body, 0)

        def scan_body(i, carry):
            b1, b2, b3 = carry
            c = NCH - 1 - i
            acc = jnp.zeros((16,), jnp.float32)
            for u in range(15, -1, -1):
                tv = _bi(c * 16 + u)
                v1 = plsc.load_gather(lc1, [tv]) + b1
                v2 = plsc.load_gather(lc2, [tv]) + b2
                v3 = plsc.load_gather(lc3, [tv]) + b3
                bn = _lse3(v1, v2, v3, ltv)
                acc = jnp.where(iota == u, bn, acc)
                b3, b2, b1 = b2, b1, bn
            plsc.store_scatter(emit, [_bi(c * 16) + iota], acc)
            return (b1, b2, b3)

        lax.fori_loop(0, NCH, scan_body, (_bf(0.0), neg16, neg16))
        emit[pl.ds(T, 16)] = jnp.where(iota == 0, np.float32(0.0), NEG)
        pltpu.sync_copy(emit, bsh)

    plsc.subcore_barrier()

    base = c_id * 2048 + s_id * 128
    s0a = pl.multiple_of(jnp.maximum(base - 8, 0), 8)
    basem = pl.multiple_of(base, 128)
    pltpu.sync_copy(ash.at[pl.ds(s0a, 160)], aloc.at[pl.ds(0, 160)])
    pltpu.sync_copy(bsh.at[pl.ds(basem, 160)], bloc.at[pl.ds(0, 160)])
    pltpu.sync_copy(ash.at[pl.ds(T - 16, 16)], nloc.at[pl.ds(0, 16)])
    normv = plsc.load_gather(nloc, [_bi(15)])
    for ch in range(8):
        tv = _bi(base + ch * 16) + iota
        codev = plsc.load_gather(codes, [tv])
        ja = jnp.maximum(tv - 1 - s0a, 0)
        av = plsc.load_gather(aloc, [ja])
        av = jnp.where(tv == 0, np.float32(0.0), av)

        def pbody(v, carry):
            vv = _bi(v)
            cv = plsc.load_gather(pcode, [vv])
            mk = plsc.load_gather(pmask, [vv])
            ln = plsc.load_gather(plenv, [vv])
            lpb = plsc.load_gather(lpv, [vv])
            match = jnp.logical_and((codev & mk) == cv, tv <= T - ln)
            bv = plsc.load_gather(bloc, [(tv + ln) - base])
            logp_tv = ((av + lpb) + bv) - normv
            pv = jnp.exp(jnp.maximum(logp_tv, CLAMP))
            pv = jnp.where(match, pv, np.float32(0.0))
            plsc.store_scatter(pblk, [(_bi(ch * 16) + iota) * 32 + vv], pv)
            return carry

        lax.fori_loop(0, 32, pbody, 0)
    ob = pl.multiple_of(basem * 32, 4096)
    pltpu.sync_copy(pblk, out_hbm.at[pl.ds(ob, 4096)])


_kernel_call = pl.kernel(
    _body,
    out_type=jax.ShapeDtypeStruct((T * 32,), jnp.float32),
    mesh=plsc.VectorSubcoreMesh(core_axis_name="c", subcore_axis_name="s"),
    compiler_params=pltpu.CompilerParams(needs_layout_passes=False),
    scratch_types=[
        pltpu.VMEM((T + 128,), jnp.int32),
        pltpu.VMEM((T,), jnp.int32),
        pltpu.VMEM((128,), jnp.float32),
        pltpu.VMEM((128,), jnp.float32),
        pltpu.VMEM((512,), jnp.float32),
        pltpu.VMEM((128,), jnp.int32),
        pltpu.VMEM((128,), jnp.int32),
        pltpu.VMEM((128,), jnp.int32),
        pltpu.VMEM((128,), jnp.float32),
        pltpu.VMEM((128,), jnp.int32),
        pltpu.VMEM((T,), jnp.float32),
        pltpu.VMEM((T,), jnp.float32),
        pltpu.VMEM((T,), jnp.float32),
        pltpu.VMEM((T + 128,), jnp.float32),
        pltpu.VMEM((256,), jnp.float32),
        pltpu.VMEM((256,), jnp.float32),
        pltpu.VMEM((128,), jnp.float32),
        pltpu.VMEM((T,), jnp.float32),
        pltpu.VMEM((1024,), jnp.float32),
        pltpu.VMEM_SHARED((T + 128,), jnp.float32),
        pltpu.VMEM_SHARED((T + 128,), jnp.float32),
    ],
)


def kernel(sequence, pieces, piece_len, log_piece_probs):
    i0 = jnp.int32(0)
    seq_pad = jnp.concatenate([sequence, jnp.full((128,), i0, sequence.dtype)])
    pieces_pad = jnp.concatenate([pieces.reshape(96), jnp.full((32,), i0)])
    plen_pad = jnp.concatenate([piece_len, jnp.full((96,), i0, piece_len.dtype)])
    logp_pad = jnp.concatenate(
        [log_piece_probs, jnp.zeros((96,), log_piece_probs.dtype)])
    out = _kernel_call(seq_pad, pieces_pad, plen_pad, logp_pad,
                       jnp.asarray(_LOGT))
    return out.reshape(T, 32)


# speedup vs baseline: 1.3245x; 1.0299x over previous
import jax
import numpy as np
import jax.numpy as jnp
from jax import lax
from jax.experimental import pallas as pl
from jax.experimental.pallas import tpu as pltpu
from jax.experimental.pallas import tpu_sc as plsc

T = 4096
NCH = T // 16
NEG = np.float32("-inf")
LN2 = np.float32(0.6931471805599453)
CLAMP = np.float32(-87.0)


def _iota():
    return lax.iota(jnp.int32, 16)


def _bi(x):
    return jnp.full((16,), x, jnp.int32)


def _bf(x):
    return jnp.full((16,), x, jnp.float32)


def _build_log_table():
    idx = np.arange(256)
    lo = (0x3F800000 + (idx << 16)).astype(np.uint32).view(np.float32)
    hi = (0x3F800000 + ((idx + 1) << 16)).astype(np.uint32).view(np.float32)
    m = ((lo.astype(np.float64) + hi.astype(np.float64)) / 2).astype(np.float32)
    md = m.astype(np.float64)
    c0 = np.log(md).astype(np.float32)
    c1 = (1.0 / md).astype(np.float32)
    c2 = (-0.5 / (md * md)).astype(np.float32)
    return np.concatenate([m, c0, c1, c2]).astype(np.float32)


_LOGT = _build_log_table()


def _log_tab(s, lt):
    sb = plsc.bitcast(s, jnp.int32)
    i0 = lax.shift_right_logical(sb + np.int32((256 << 16) - 0x3F800000), 16)
    m = plsc.bitcast((sb & np.int32(~0xFFFF)) | np.int32(0x8000), jnp.float32)
    c0 = plsc.load_gather(lt, [i0])
    c1 = plsc.load_gather(lt, [i0 + 256])
    c2 = plsc.load_gather(lt, [i0 + 512])
    x = s - m
    return ((c2 * x) * x + (c1 * x + c0))


def _log_1_3(s):
    big = s >= 1.5
    h = jnp.where(big, s * 0.5, s)
    z = (h - 1.0) / (h + 1.0)
    w = z * z
    p = w * np.float32(1.0 / 9.0) + np.float32(1.0 / 7.0)
    p = p * w + np.float32(1.0 / 5.0)
    p = p * w + np.float32(1.0 / 3.0)
    p = p * w + np.float32(1.0)
    r = (z + z) * p
    return jnp.where(big, r + LN2, r)


def _lse3(v1, v2, v3, lt):
    mx = jnp.maximum(v1, jnp.maximum(v2, v3))
    e1 = jnp.exp(v1 - mx)
    e2 = jnp.exp(jnp.maximum(v2 - mx, CLAMP))
    e3 = jnp.exp(jnp.maximum(v3 - mx, CLAMP))
    return mx + _log_tab((e1 + e2) + e3, lt)


def _body(seq_hbm, pieces_hbm, plen_hbm, logp_hbm, logt_hbm, out_hbm,
          seqv, tb1, tb2, tb3, pcode, pmask, plenv, lpv, piecv,
          lc1, lc2, lc3, emit, aloc, bloc, nloc, pblk, ltv, ash, bsh):
    iota = _iota()
    s_id = lax.axis_index("s")
    c_id = lax.axis_index("c")

    pltpu.sync_copy(seq_hbm, seqv)
    pltpu.sync_copy(pieces_hbm, piecv)
    pltpu.sync_copy(plen_hbm, plenv)
    pltpu.sync_copy(logp_hbm, lpv)
    pltpu.sync_copy(logt_hbm, ltv)

    neg16 = _bf(NEG)
    for k in range(8):
        tb1[pl.ds(k * 16, 16)] = neg16
        tb2[pl.ds(k * 16, 16)] = neg16
    for k in range(32):
        tb3[pl.ds(k * 16, 16)] = neg16
    for h in range(2):
        l = plenv[pl.ds(h * 16, 16)]
        lp = lpv[pl.ds(h * 16, 16)]
        r3 = (_bi(h * 16) + iota) * 3
        p0 = plsc.load_gather(piecv, [r3])
        p1 = plsc.load_gather(piecv, [r3 + 1])
        p2 = plsc.load_gather(piecv, [r3 + 2])
        code = (p0 + jnp.where(l >= 2, p1 * 8, 0)) + jnp.where(l >= 3, p2 * 64, 0)
        mask = (_bi(1) << (l * 3)) - 1
        pcode[pl.ds(h * 16, 16)] = code
        pmask[pl.ds(h * 16, 16)] = mask
        plsc.store_scatter(tb1, [code & 7], lp, mask=l == 1)
        plsc.store_scatter(tb2, [code & 63], lp, mask=l == 2)
        plsc.store_scatter(tb3, [code & 511], lp, mask=l == 3)

    @pl.when(s_id == 0)
    def _():
        def lc_body(i, carry):
            j = _bi(i * 16) + iota
            gm2 = plsc.load_gather(seqv, [jnp.maximum(j - 2, 0)])
            gm1 = plsc.load_gather(seqv, [jnp.maximum(j - 1, 0)])
            g0 = plsc.load_gather(seqv, [j])
            g1 = plsc.load_gather(seqv, [j + 1])
            g2 = plsc.load_gather(seqv, [j + 2])
            plsc.store_scatter(lc1, [j], plsc.load_gather(tb1, [g0]))
            c2 = gm1 + 8 * g0
            v2 = plsc.load_gather(tb2, [c2])
            plsc.store_scatter(lc2, [j], jnp.where(j >= 1, v2, NEG))
            c3 = (gm2 + 8 * gm1) + 64 * g0
            v3 = plsc.load_gather(tb3, [c3])
            plsc.store_scatter(lc3, [j], jnp.where(j >= 2, v3, NEG))
            return carry

        lax.fori_loop(0, NCH, lc_body, 0)

        def scan_body(i, carry):
            a1, a2, a3 = carry
            acc = jnp.zeros((16,), jnp.float32)
            for u in range(16):
                jv = _bi(i * 16 + u)
                v1 = a1 + plsc.load_gather(lc1, [jv])
                v2 = a2 + plsc.load_gather(lc2, [jv])
                v3 = a3 + plsc.load_gather(lc3, [jv])
                an = _lse3(v1, v2, v3, ltv)
                acc = jnp.where(iota == u, an, acc)
                a3, a2, a1 = a2, a1, an
            plsc.store_scatter(emit, [_bi(i * 16) + iota], acc)
            return (a1, a2, a3)

        lax.fori_loop(0, NCH, scan_body, (_bf(0.0), neg16, neg16))
        pltpu.sync_copy(emit, ash)

    @pl.when(s_id == 1)
    def _():
        def lc_body(i, carry):
            t = _bi(i * 16) + iota
            g0 = plsc.load_gather(seqv, [t])
            g1 = plsc.load_gather(seqv, [t + 1])
            g2 = plsc.load_gather(seqv, [t + 2])
            plsc.store_scatter(lc1, [t], plsc.load_gather(tb1, [g0]))
            v2 = plsc.load_gather(tb2, [g0 + 8 * g1])
            plsc.store_scatter(lc2, [t], jnp.where(t <= T - 2, v2, NEG))
            v3 = plsc.load_gather(tb3, [(g0 + 8 * g1) + 64 * g2])
            plsc.store_scatter(lc3, [t], jnp.where(t <= T - 3, v3, NEG))
            return carry

        lax.fori_loop(0, NCH, lc_body, 0)

        def scan_body(i, carry):
            b1, b2, b3 = carry
            c = NCH - 1 - i
            acc = jnp.zeros((16,), jnp.float32)
            for u in range(15, -1, -1):
                tv = _bi(c * 16 + u)
                v1 = plsc.load_gather(lc1, [tv]) + b1
                v2 = plsc.load_gather(lc2, [tv]) + b2
                v3 = plsc.load_gather(lc3, [tv]) + b3
                bn = _lse3(v1, v2, v3, ltv)
                acc = jnp.where(iota == u, bn, acc)
                b3, b2, b1 = b2, b1, bn
            plsc.store_scatter(emit, [_bi(c * 16) + iota], acc)
            return (b1, b2, b3)

        lax.fori_loop(0, NCH, scan_body, (_bf(0.0), neg16, neg16))
        emit[pl.ds(T, 16)] = jnp.where(iota == 0, np.float32(0.0), NEG)
        pltpu.sync_copy(emit, bsh)

    plsc.subcore_barrier()

    base = c_id * 2048 + s_id * 128
    s0a = pl.multiple_of(jnp.maximum(base - 8, 0), 8)
    basem = pl.multiple_of(base, 128)
    pltpu.sync_copy(ash.at[pl.ds(s0a, 160)], aloc.at[pl.ds(0, 160)])
    pltpu.sync_copy(bsh.at[pl.ds(basem, 160)], bloc.at[pl.ds(0, 160)])
    pltpu.sync_copy(ash.at[pl.ds(T - 16, 16)], nloc.at[pl.ds(0, 16)])
    normv = plsc.load_gather(nloc, [_bi(15)])
    for ch in range(8):
        tv = _bi(base + ch * 16) + iota
        g0 = plsc.load_gather(seqv, [tv])
        g1 = plsc.load_gather(seqv, [tv + 1])
        g2 = plsc.load_gather(seqv, [tv + 2])
        codev = (g0 + 8 * g1) + 64 * g2
        ja = jnp.maximum(tv - 1 - s0a, 0)
        av = plsc.load_gather(aloc, [ja])
        av = jnp.where(tv == 0, np.float32(0.0), av)

        def pbody(v, carry):
            vv = _bi(v)
            cv = plsc.load_gather(pcode, [vv])
            mk = plsc.load_gather(pmask, [vv])
            ln = plsc.load_gather(plenv, [vv])
            lpb = plsc.load_gather(lpv, [vv])
            match = jnp.logical_and((codev & mk) == cv, tv <= T - ln)
            bv = plsc.load_gather(bloc, [(tv + ln) - base])
            logp_tv = ((av + lpb) + bv) - normv
            pv = jnp.exp(jnp.maximum(logp_tv, CLAMP))
            pv = jnp.where(match, pv, np.float32(0.0))
            plsc.store_scatter(pblk, [(_bi(ch * 16) + iota) * 32 + vv], pv)
            return carry

        lax.fori_loop(0, 32, pbody, 0)
    ob = pl.multiple_of(basem * 32, 4096)
    pltpu.sync_copy(pblk, out_hbm.at[pl.ds(ob, 4096)])


_kernel_call = pl.kernel(
    _body,
    out_type=jax.ShapeDtypeStruct((T * 32,), jnp.float32),
    mesh=plsc.VectorSubcoreMesh(core_axis_name="c", subcore_axis_name="s"),
    compiler_params=pltpu.CompilerParams(needs_layout_passes=False),
    scratch_types=[
        pltpu.VMEM((T + 128,), jnp.int32),
        pltpu.VMEM((128,), jnp.float32),
        pltpu.VMEM((128,), jnp.float32),
        pltpu.VMEM((512,), jnp.float32),
        pltpu.VMEM((128,), jnp.int32),
        pltpu.VMEM((128,), jnp.int32),
        pltpu.VMEM((128,), jnp.int32),
        pltpu.VMEM((128,), jnp.float32),
        pltpu.VMEM((128,), jnp.int32),
        pltpu.VMEM((T,), jnp.float32),
        pltpu.VMEM((T,), jnp.float32),
        pltpu.VMEM((T,), jnp.float32),
        pltpu.VMEM((T + 128,), jnp.float32),
        pltpu.VMEM((256,), jnp.float32),
        pltpu.VMEM((256,), jnp.float32),
        pltpu.VMEM((128,), jnp.float32),
        pltpu.VMEM((T,), jnp.float32),
        pltpu.VMEM((1024,), jnp.float32),
        pltpu.VMEM_SHARED((T + 128,), jnp.float32),
        pltpu.VMEM_SHARED((T + 128,), jnp.float32),
    ],
)


def kernel(sequence, pieces, piece_len, log_piece_probs):
    i0 = jnp.int32(0)
    seq_pad = jnp.concatenate([sequence, jnp.full((128,), i0, sequence.dtype)])
    pieces_pad = jnp.concatenate([pieces.reshape(96), jnp.full((32,), i0)])
    plen_pad = jnp.concatenate([piece_len, jnp.full((96,), i0, piece_len.dtype)])
    logp_pad = jnp.concatenate(
        [log_piece_probs, jnp.zeros((96,), log_piece_probs.dtype)])
    out = _kernel_call(seq_pad, pieces_pad, plen_pad, logp_pad,
                       jnp.asarray(_LOGT))
    return out.reshape(T, 32)


# speedup vs baseline: 1.3456x; 1.0160x over previous
import jax
import numpy as np
import jax.numpy as jnp
from jax import lax
from jax.experimental import pallas as pl
from jax.experimental.pallas import tpu as pltpu
from jax.experimental.pallas import tpu_sc as plsc

T = 4096
NCH = T // 16
NEG = np.float32("-inf")
LN2 = np.float32(0.6931471805599453)
CLAMP = np.float32(-87.0)


def _iota():
    return lax.iota(jnp.int32, 16)


def _bi(x):
    return jnp.full((16,), x, jnp.int32)


def _bf(x):
    return jnp.full((16,), x, jnp.float32)


def _build_log_table():
    idx = np.arange(256)
    lo = (0x3F800000 + (idx << 16)).astype(np.uint32).view(np.float32)
    hi = (0x3F800000 + ((idx + 1) << 16)).astype(np.uint32).view(np.float32)
    m = ((lo.astype(np.float64) + hi.astype(np.float64)) / 2).astype(np.float32)
    md = m.astype(np.float64)
    c0 = np.log(md).astype(np.float32)
    c1 = (1.0 / md).astype(np.float32)
    c2 = (-0.5 / (md * md)).astype(np.float32)
    return np.concatenate([m, c0, c1, c2]).astype(np.float32)


_LOGT = _build_log_table()


def _log_tab(s, lt):
    sb = plsc.bitcast(s, jnp.int32)
    i0 = lax.shift_right_logical(sb + np.int32((256 << 16) - 0x3F800000), 16)
    m = plsc.bitcast((sb & np.int32(~0xFFFF)) | np.int32(0x8000), jnp.float32)
    c0 = plsc.load_gather(lt, [i0])
    c1 = plsc.load_gather(lt, [i0 + 256])
    c2 = plsc.load_gather(lt, [i0 + 512])
    x = s - m
    return ((c2 * x) * x + (c1 * x + c0))


def _log_1_3(s):
    big = s >= 1.5
    h = jnp.where(big, s * 0.5, s)
    z = (h - 1.0) / (h + 1.0)
    w = z * z
    p = w * np.float32(1.0 / 9.0) + np.float32(1.0 / 7.0)
    p = p * w + np.float32(1.0 / 5.0)
    p = p * w + np.float32(1.0 / 3.0)
    p = p * w + np.float32(1.0)
    r = (z + z) * p
    return jnp.where(big, r + LN2, r)


def _lse3(v1, v2, v3, lt):
    mx = jnp.maximum(v1, jnp.maximum(v2, v3))
    e1 = jnp.exp(v1 - mx)
    e2 = jnp.exp(jnp.maximum(v2 - mx, CLAMP))
    e3 = jnp.exp(jnp.maximum(v3 - mx, CLAMP))
    return mx + _log_tab((e1 + e2) + e3, lt)


def _body(seq_hbm, pieces_hbm, plen_hbm, logp_hbm, logt_hbm, out_hbm,
          seqv, tb1, tb2, tb3, pcode, pmask, plenv, lpv, piecv,
          lc1, lc2, lc3, emit, aloc, bloc, nloc, pblk, ltv, ash, bsh):
    iota = _iota()
    s_id = lax.axis_index("s")
    c_id = lax.axis_index("c")

    pltpu.sync_copy(seq_hbm, seqv.at[pl.ds(0, T)])
    seqv[pl.ds(T, 16)] = jnp.zeros((16,), jnp.int32)
    pltpu.sync_copy(pieces_hbm, piecv.at[pl.ds(0, 96)])
    pltpu.sync_copy(plen_hbm, plenv.at[pl.ds(0, 32)])
    pltpu.sync_copy(logp_hbm, lpv.at[pl.ds(0, 32)])
    pltpu.sync_copy(logt_hbm, ltv)

    neg16 = _bf(NEG)
    for k in range(8):
        tb1[pl.ds(k * 16, 16)] = neg16
        tb2[pl.ds(k * 16, 16)] = neg16
    for k in range(32):
        tb3[pl.ds(k * 16, 16)] = neg16
    for h in range(2):
        l = plenv[pl.ds(h * 16, 16)]
        lp = lpv[pl.ds(h * 16, 16)]
        r3 = (_bi(h * 16) + iota) * 3
        p0 = plsc.load_gather(piecv, [r3])
        p1 = plsc.load_gather(piecv, [r3 + 1])
        p2 = plsc.load_gather(piecv, [r3 + 2])
        code = (p0 + jnp.where(l >= 2, p1 * 8, 0)) + jnp.where(l >= 3, p2 * 64, 0)
        mask = (_bi(1) << (l * 3)) - 1
        pcode[pl.ds(h * 16, 16)] = code
        pmask[pl.ds(h * 16, 16)] = mask
        plsc.store_scatter(tb1, [code & 7], lp, mask=l == 1)
        plsc.store_scatter(tb2, [code & 63], lp, mask=l == 2)
        plsc.store_scatter(tb3, [code & 511], lp, mask=l == 3)

    @pl.when(s_id == 0)
    def _():
        def lc_body(i, carry):
            j = _bi(i * 16) + iota
            gm2 = plsc.load_gather(seqv, [jnp.maximum(j - 2, 0)])
            gm1 = plsc.load_gather(seqv, [jnp.maximum(j - 1, 0)])
            g0 = plsc.load_gather(seqv, [j])
            g1 = plsc.load_gather(seqv, [j + 1])
            g2 = plsc.load_gather(seqv, [j + 2])
            plsc.store_scatter(lc1, [j], plsc.load_gather(tb1, [g0]))
            c2 = gm1 + 8 * g0
            v2 = plsc.load_gather(tb2, [c2])
            plsc.store_scatter(lc2, [j], jnp.where(j >= 1, v2, NEG))
            c3 = (gm2 + 8 * gm1) + 64 * g0
            v3 = plsc.load_gather(tb3, [c3])
            plsc.store_scatter(lc3, [j], jnp.where(j >= 2, v3, NEG))
            return carry

        lax.fori_loop(0, NCH, lc_body, 0)

        def scan_body(i, carry):
            a1, a2, a3 = carry
            acc = jnp.zeros((16,), jnp.float32)
            for u in range(16):
                jv = _bi(i * 16 + u)
                v1 = a1 + plsc.load_gather(lc1, [jv])
                v2 = a2 + plsc.load_gather(lc2, [jv])
                v3 = a3 + plsc.load_gather(lc3, [jv])
                an = _lse3(v1, v2, v3, ltv)
                acc = jnp.where(iota == u, an, acc)
                a3, a2, a1 = a2, a1, an
            plsc.store_scatter(emit, [_bi(i * 16) + iota], acc)
            return (a1, a2, a3)

        lax.fori_loop(0, NCH, scan_body, (_bf(0.0), neg16, neg16))
        pltpu.sync_copy(emit, ash)

    @pl.when(s_id == 1)
    def _():
        def lc_body(i, carry):
            t = _bi(i * 16) + iota
            g0 = plsc.load_gather(seqv, [t])
            g1 = plsc.load_gather(seqv, [t + 1])
            g2 = plsc.load_gather(seqv, [t + 2])
            plsc.store_scatter(lc1, [t], plsc.load_gather(tb1, [g0]))
            v2 = plsc.load_gather(tb2, [g0 + 8 * g1])
            plsc.store_scatter(lc2, [t], jnp.where(t <= T - 2, v2, NEG))
            v3 = plsc.load_gather(tb3, [(g0 + 8 * g1) + 64 * g2])
            plsc.store_scatter(lc3, [t], jnp.where(t <= T - 3, v3, NEG))
            return carry

        lax.fori_loop(0, NCH, lc_body, 0)

        def scan_body(i, carry):
            b1, b2, b3 = carry
            c = NCH - 1 - i
            acc = jnp.zeros((16,), jnp.float32)
            for u in range(15, -1, -1):
                tv = _bi(c * 16 + u)
                v1 = plsc.load_gather(lc1, [tv]) + b1
                v2 = plsc.load_gather(lc2, [tv]) + b2
                v3 = plsc.load_gather(lc3, [tv]) + b3
                bn = _lse3(v1, v2, v3, ltv)
                acc = jnp.where(iota == u, bn, acc)
                b3, b2, b1 = b2, b1, bn
            plsc.store_scatter(emit, [_bi(c * 16) + iota], acc)
            return (b1, b2, b3)

        lax.fori_loop(0, NCH, scan_body, (_bf(0.0), neg16, neg16))
        emit[pl.ds(T, 16)] = jnp.where(iota == 0, np.float32(0.0), NEG)
        pltpu.sync_copy(emit, bsh)

    plsc.subcore_barrier()

    base = c_id * 2048 + s_id * 128
    s0a = pl.multiple_of(jnp.maximum(base - 8, 0), 8)
    basem = pl.multiple_of(base, 128)
    pltpu.sync_copy(ash.at[pl.ds(s0a, 160)], aloc.at[pl.ds(0, 160)])
    pltpu.sync_copy(bsh.at[pl.ds(basem, 160)], bloc.at[pl.ds(0, 160)])
    pltpu.sync_copy(ash.at[pl.ds(T - 16, 16)], nloc.at[pl.ds(0, 16)])
    normv = plsc.load_gather(nloc, [_bi(15)])
    for ch in range(8):
        tv = _bi(base + ch * 16) + iota
        g0 = plsc.load_gather(seqv, [tv])
        g1 = plsc.load_gather(seqv, [tv + 1])
        g2 = plsc.load_gather(seqv, [tv + 2])
        codev = (g0 + 8 * g1) + 64 * g2
        ja = jnp.maximum(tv - 1 - s0a, 0)
        av = plsc.load_gather(aloc, [ja])
        av = jnp.where(tv == 0, np.float32(0.0), av)

        def pbody(v, carry):
            vv = _bi(v)
            cv = plsc.load_gather(pcode, [vv])
            mk = plsc.load_gather(pmask, [vv])
            ln = plsc.load_gather(plenv, [vv])
            lpb = plsc.load_gather(lpv, [vv])
            match = jnp.logical_and((codev & mk) == cv, tv <= T - ln)
            bv = plsc.load_gather(bloc, [(tv + ln) - base])
            logp_tv = ((av + lpb) + bv) - normv
            pv = jnp.exp(jnp.maximum(logp_tv, CLAMP))
            pv = jnp.where(match, pv, np.float32(0.0))
            plsc.store_scatter(pblk, [(_bi(ch * 16) + iota) * 32 + vv], pv)
            return carry

        lax.fori_loop(0, 32, pbody, 0)
    ob = pl.multiple_of(basem * 32, 4096)
    pltpu.sync_copy(pblk, out_hbm.at[pl.ds(ob, 4096)])


_kernel_call = pl.kernel(
    _body,
    out_type=jax.ShapeDtypeStruct((T * 32,), jnp.float32),
    mesh=plsc.VectorSubcoreMesh(core_axis_name="c", subcore_axis_name="s"),
    compiler_params=pltpu.CompilerParams(needs_layout_passes=False),
    scratch_types=[
        pltpu.VMEM((T + 128,), jnp.int32),
        pltpu.VMEM((128,), jnp.float32),
        pltpu.VMEM((128,), jnp.float32),
        pltpu.VMEM((512,), jnp.float32),
        pltpu.VMEM((128,), jnp.int32),
        pltpu.VMEM((128,), jnp.int32),
        pltpu.VMEM((128,), jnp.int32),
        pltpu.VMEM((128,), jnp.float32),
        pltpu.VMEM((128,), jnp.int32),
        pltpu.VMEM((T,), jnp.float32),
        pltpu.VMEM((T,), jnp.float32),
        pltpu.VMEM((T,), jnp.float32),
        pltpu.VMEM((T + 128,), jnp.float32),
        pltpu.VMEM((256,), jnp.float32),
        pltpu.VMEM((256,), jnp.float32),
        pltpu.VMEM((128,), jnp.float32),
        pltpu.VMEM((T,), jnp.float32),
        pltpu.VMEM((1024,), jnp.float32),
        pltpu.VMEM_SHARED((T + 128,), jnp.float32),
        pltpu.VMEM_SHARED((T + 128,), jnp.float32),
    ],
)


def kernel(sequence, pieces, piece_len, log_piece_probs):
    out = _kernel_call(sequence, pieces.reshape(96), piece_len,
                       log_piece_probs, jnp.asarray(_LOGT))
    return out.reshape(T, 32)


# speedup vs baseline: 1.3471x; 1.0012x over previous
import jax
import numpy as np
import jax.numpy as jnp
from jax import lax
from jax.experimental import pallas as pl
from jax.experimental.pallas import tpu as pltpu
from jax.experimental.pallas import tpu_sc as plsc

T = 4096
NCH = T // 16
NEG = np.float32("-inf")
CLAMP = np.float32(-87.0)


def _iota():
    return lax.iota(jnp.int32, 16)


def _bi(x):
    return jnp.full((16,), x, jnp.int32)


def _bf(x):
    return jnp.full((16,), x, jnp.float32)


def _build_log_table():
    idx = np.arange(256)
    lo = (0x3F800000 + (idx << 16)).astype(np.uint32).view(np.float32)
    hi = (0x3F800000 + ((idx + 1) << 16)).astype(np.uint32).view(np.float32)
    m = ((lo.astype(np.float64) + hi.astype(np.float64)) / 2).astype(np.float32)
    md = m.astype(np.float64)
    c0 = np.log(md).astype(np.float32)
    c1 = (1.0 / md).astype(np.float32)
    c2 = (-0.5 / (md * md)).astype(np.float32)
    return np.concatenate([m, c0, c1, c2]).astype(np.float32)


_LOGT = _build_log_table()


def _log_tab(s, lt):
    sb = plsc.bitcast(s, jnp.int32)
    i0 = lax.shift_right_logical(sb + np.int32((256 << 16) - 0x3F800000), 16)
    m = plsc.bitcast((sb & np.int32(~0xFFFF)) | np.int32(0x8000), jnp.float32)
    c0 = plsc.load_gather(lt, [i0])
    c1 = plsc.load_gather(lt, [i0 + 256])
    c2 = plsc.load_gather(lt, [i0 + 512])
    x = s - m
    return ((c2 * x) * x + (c1 * x + c0))


def _lse3(v1, v2, v3, lt):
    mx = jnp.maximum(v1, jnp.maximum(v2, v3))
    e1 = jnp.exp(v1 - mx)
    e2 = jnp.exp(jnp.maximum(v2 - mx, CLAMP))
    e3 = jnp.exp(jnp.maximum(v3 - mx, CLAMP))
    return mx + _log_tab((e1 + e2) + e3, lt)


def _body(seq_hbm, pieces_hbm, plen_hbm, logp_hbm, logt_hbm, out_hbm,
          seqv, tb1, tb2, tb3, pcode, pmask, plenv, lpv, piecv,
          lc1, lc2, lc3, emit, aloc, bloc, nloc, pblk, ltv, ash, bsh):
    iota = _iota()
    s_id = lax.axis_index("s")
    c_id = lax.axis_index("c")

    pltpu.sync_copy(seq_hbm, seqv.at[pl.ds(0, T)])
    seqv[pl.ds(T, 16)] = jnp.zeros((16,), jnp.int32)
    pltpu.sync_copy(pieces_hbm, piecv.at[pl.ds(0, 96)])
    pltpu.sync_copy(plen_hbm, plenv.at[pl.ds(0, 32)])
    pltpu.sync_copy(logp_hbm, lpv.at[pl.ds(0, 32)])
    pltpu.sync_copy(logt_hbm, ltv)

    neg16 = _bf(NEG)
    for k in range(8):
        tb1[pl.ds(k * 16, 16)] = neg16
        tb2[pl.ds(k * 16, 16)] = neg16
    for k in range(32):
        tb3[pl.ds(k * 16, 16)] = neg16
    for h in range(2):
        l = plenv[pl.ds(h * 16, 16)]
        lp = lpv[pl.ds(h * 16, 16)]
        r3 = (_bi(h * 16) + iota) * 3
        p0 = plsc.load_gather(piecv, [r3])
        p1 = plsc.load_gather(piecv, [r3 + 1])
        p2 = plsc.load_gather(piecv, [r3 + 2])
        code = (p0 + jnp.where(l >= 2, p1 * 8, 0)) + jnp.where(l >= 3, p2 * 64, 0)
        mask = (_bi(1) << (l * 3)) - 1
        pcode[pl.ds(h * 16, 16)] = code
        pmask[pl.ds(h * 16, 16)] = mask
        plsc.store_scatter(tb1, [code & 7], lp, mask=l == 1)
        plsc.store_scatter(tb2, [code & 63], lp, mask=l == 2)
        plsc.store_scatter(tb3, [code & 511], lp, mask=l == 3)

    @pl.when(s_id == 0)
    def _():
        def lc_body(i, carry):
            j = _bi(i * 16) + iota
            gm2 = plsc.load_gather(seqv, [jnp.maximum(j - 2, 0)])
            gm1 = plsc.load_gather(seqv, [jnp.maximum(j - 1, 0)])
            g0 = plsc.load_gather(seqv, [j])
            g1 = plsc.load_gather(seqv, [j + 1])
            g2 = plsc.load_gather(seqv, [j + 2])
            plsc.store_scatter(lc1, [j], plsc.load_gather(tb1, [g0]))
            c2 = gm1 + 8 * g0
            v2 = plsc.load_gather(tb2, [c2])
            plsc.store_scatter(lc2, [j], jnp.where(j >= 1, v2, NEG))
            c3 = (gm2 + 8 * gm1) + 64 * g0
            v3 = plsc.load_gather(tb3, [c3])
            plsc.store_scatter(lc3, [j], jnp.where(j >= 2, v3, NEG))
            return carry

        lax.fori_loop(0, NCH, lc_body, 0)

        def scan_body(i, carry):
            a1, a2, a3 = carry
            acc = jnp.zeros((16,), jnp.float32)
            for u in range(16):
                jv = _bi(i * 16 + u)
                v1 = a1 + plsc.load_gather(lc1, [jv])
                v2 = a2 + plsc.load_gather(lc2, [jv])
                v3 = a3 + plsc.load_gather(lc3, [jv])
                an = _lse3(v1, v2, v3, ltv)
                acc = jnp.where(iota == u, an, acc)
                a3, a2, a1 = a2, a1, an
            plsc.store_scatter(emit, [_bi(i * 16) + iota], acc)
            return (a1, a2, a3)

        lax.fori_loop(0, NCH, scan_body, (_bf(0.0), neg16, neg16))
        pltpu.sync_copy(emit, ash)

    @pl.when(s_id == 1)
    def _():
        def lc_body(i, carry):
            t = _bi(i * 16) + iota
            g0 = plsc.load_gather(seqv, [t])
            g1 = plsc.load_gather(seqv, [t + 1])
            g2 = plsc.load_gather(seqv, [t + 2])
            plsc.store_scatter(lc1, [t], plsc.load_gather(tb1, [g0]))
            v2 = plsc.load_gather(tb2, [g0 + 8 * g1])
            plsc.store_scatter(lc2, [t], jnp.where(t <= T - 2, v2, NEG))
            v3 = plsc.load_gather(tb3, [(g0 + 8 * g1) + 64 * g2])
            plsc.store_scatter(lc3, [t], jnp.where(t <= T - 3, v3, NEG))
            return carry

        lax.fori_loop(0, NCH, lc_body, 0)

        def scan_body(i, carry):
            b1, b2, b3 = carry
            c = NCH - 1 - i
            acc = jnp.zeros((16,), jnp.float32)
            for u in range(15, -1, -1):
                tv = _bi(c * 16 + u)
                v1 = plsc.load_gather(lc1, [tv]) + b1
                v2 = plsc.load_gather(lc2, [tv]) + b2
                v3 = plsc.load_gather(lc3, [tv]) + b3
                bn = _lse3(v1, v2, v3, ltv)
                acc = jnp.where(iota == u, bn, acc)
                b3, b2, b1 = b2, b1, bn
            plsc.store_scatter(emit, [_bi(c * 16) + iota], acc)
            return (b1, b2, b3)

        lax.fori_loop(0, NCH, scan_body, (_bf(0.0), neg16, neg16))
        emit[pl.ds(T, 16)] = jnp.where(iota == 0, np.float32(0.0), NEG)
        pltpu.sync_copy(emit, bsh)

    plsc.subcore_barrier()

    base = c_id * 2048 + s_id * 128
    s0a = pl.multiple_of(jnp.maximum(base - 8, 0), 8)
    basem = pl.multiple_of(base, 128)
    pltpu.sync_copy(ash.at[pl.ds(s0a, 160)], aloc.at[pl.ds(0, 160)])
    pltpu.sync_copy(bsh.at[pl.ds(basem, 160)], bloc.at[pl.ds(0, 160)])
    pltpu.sync_copy(ash.at[pl.ds(T - 16, 16)], nloc.at[pl.ds(0, 16)])
    normv = plsc.load_gather(nloc, [_bi(15)])
    for ch in range(8):
        tv = _bi(base + ch * 16) + iota
        g0 = plsc.load_gather(seqv, [tv])
        g1 = plsc.load_gather(seqv, [tv + 1])
        g2 = plsc.load_gather(seqv, [tv + 2])
        codev = (g0 + 8 * g1) + 64 * g2
        ja = jnp.maximum(tv - 1 - s0a, 0)
        av = plsc.load_gather(aloc, [ja])
        av = jnp.where(tv == 0, np.float32(0.0), av)

        def pbody(v, carry):
            vv = _bi(v)
            cv = plsc.load_gather(pcode, [vv])
            mk = plsc.load_gather(pmask, [vv])
            ln = plsc.load_gather(plenv, [vv])
            lpb = plsc.load_gather(lpv, [vv])
            match = jnp.logical_and((codev & mk) == cv, tv <= T - ln)
            bv = plsc.load_gather(bloc, [(tv + ln) - base])
            logp_tv = ((av + lpb) + bv) - normv
            pv = jnp.exp(jnp.maximum(logp_tv, CLAMP))
            pv = jnp.where(match, pv, np.float32(0.0))
            plsc.store_scatter(pblk, [(_bi(ch * 16) + iota) * 32 + vv], pv)
            return carry

        lax.fori_loop(0, 32, pbody, 0)
    ob = pl.multiple_of(basem * 32, 4096)
    pltpu.sync_copy(pblk, out_hbm.at[pl.ds(ob, 4096)])


_kernel_call = pl.kernel(
    _body,
    out_type=jax.ShapeDtypeStruct((T * 32,), jnp.float32),
    mesh=plsc.VectorSubcoreMesh(core_axis_name="c", subcore_axis_name="s"),
    compiler_params=pltpu.CompilerParams(needs_layout_passes=False),
    scratch_types=[
        pltpu.VMEM((T + 128,), jnp.int32),
        pltpu.VMEM((128,), jnp.float32),
        pltpu.VMEM((128,), jnp.float32),
        pltpu.VMEM((512,), jnp.float32),
        pltpu.VMEM((128,), jnp.int32),
        pltpu.VMEM((128,), jnp.int32),
        pltpu.VMEM((128,), jnp.int32),
        pltpu.VMEM((128,), jnp.float32),
        pltpu.VMEM((128,), jnp.int32),
        pltpu.VMEM((T,), jnp.float32),
        pltpu.VMEM((T,), jnp.float32),
        pltpu.VMEM((T,), jnp.float32),
        pltpu.VMEM((T + 128,), jnp.float32),
        pltpu.VMEM((256,), jnp.float32),
        pltpu.VMEM((256,), jnp.float32),
        pltpu.VMEM((128,), jnp.float32),
        pltpu.VMEM((T,), jnp.float32),
        pltpu.VMEM((1024,), jnp.float32),
        pltpu.VMEM_SHARED((T + 128,), jnp.float32),
        pltpu.VMEM_SHARED((T + 128,), jnp.float32),
    ],
)


def kernel(sequence, pieces, piece_len, log_piece_probs):
    out = _kernel_call(sequence, pieces.reshape(96), piece_len,
                       log_piece_probs, jnp.asarray(_LOGT))
    return out.reshape(T, 32)
